# Initial kernel scaffold; baseline (speedup 1.0000x reference)
import jax
import jax.numpy as jnp
from jax.experimental import pallas as pl


def kernel(x, W_orig, b_orig, W_policy, b_policy, ground_truth, sel_idx):
    raise NotImplementedError("write your pallas kernel here")


# speedup vs baseline: 2.0780x; 2.0780x over previous
import functools
import math

import jax
import jax.numpy as jnp
from jax import lax
from jax.experimental import pallas as pl
from jax.experimental.pallas import tpu as pltpu
from jax.experimental.pallas import tpu_sc as plsc

_QUANTILE = 0.7


def _build_is_sel(sel_idx, F):
    K = sel_idx.shape[0]
    mesh = plsc.VectorSubcoreMesh(core_axis_name="c", subcore_axis_name="s")

    @functools.partial(
        pl.kernel,
        mesh=mesh,
        compiler_params=pltpu.CompilerParams(needs_layout_passes=False),
        out_type=jax.ShapeDtypeStruct((F,), jnp.int32),
        scratch_types=[
            pltpu.VMEM((K,), jnp.int32),
            pltpu.VMEM((F,), jnp.int32),
        ],
    )
    def sc_scatter(idx_hbm, out_hbm, idx_v, flags_v):
        c = lax.axis_index("c")
        s = lax.axis_index("s")

        @pl.when(jnp.logical_and(c == 0, s == 0))
        def _():
            pltpu.sync_copy(idx_hbm, idx_v)
            zeros = jnp.zeros((16,), jnp.int32)

            def zbody(i, carry):
                flags_v[pl.ds(i * 16, 16)] = zeros
                return carry

            lax.fori_loop(0, F // 16, zbody, 0)
            ones = jnp.ones((16,), jnp.int32)

            def sbody(i, carry):
                iv = idx_v[pl.ds(i * 16, 16)]
                plsc.store_scatter(flags_v, [iv], ones)
                return carry

            lax.fori_loop(0, K // 16, sbody, 0)
            pltpu.sync_copy(flags_v, out_hbm)

    return sc_scatter(sel_idx)


def _mm_bias_body(x_ref, w_ref, b_ref, o_ref):
    o_ref[...] = (
        jnp.dot(x_ref[...], w_ref[...], preferred_element_type=jnp.float32)
        + b_ref[...]
    )


def _policy_body(y_ref, w_ref, b_ref, sel_ref, o_ref):
    z = (
        jnp.dot(y_ref[...], w_ref[...], preferred_element_type=jnp.float32)
        + b_ref[...]
    )
    bits = lax.bitcast_convert_type(z, jnp.int32)
    skey = jnp.where(bits < 0, bits ^ jnp.int32(0x7FFFFFFF), bits)
    o_ref[...] = jnp.where(sel_ref[...] != 0, skey, jnp.int32(2**31 - 1))


def _select_blend_body(sk_ref, y_ref, gt_ref, o_ref, *, kth):
    sk = sk_ref[...]
    rows = sk.shape[0]
    P0 = jnp.full((rows, 1), jnp.int32(-(2**31)))

    def body(i, P):
        T = P + (jnp.int32(1) << (jnp.int32(31) - i.astype(jnp.int32)))
        cnt = jnp.sum((sk < T).astype(jnp.int32), axis=1, keepdims=True)
        return jnp.where(cnt >= kth, P, T)

    P = lax.fori_loop(0, 32, body, P0)
    keep = sk > P
    o_ref[...] = jnp.where(keep, y_ref[...], gt_ref[...])


def kernel(x, W_orig, b_orig, W_policy, b_policy, ground_truth, sel_idx):
    B, D = x.shape
    F = W_orig.shape[1]
    K = sel_idx.shape[0]
    kth = int(max(1, min(K, 1 + math.floor(_QUANTILE * (K - 1)))))
    BF = 1024
    nblk = F // BF

    is_sel = _build_is_sel(sel_idx, F).reshape(1, F)

    y = pl.pallas_call(
        _mm_bias_body,
        grid=(nblk,),
        in_specs=[
            pl.BlockSpec((B, D), lambda j: (0, 0)),
            pl.BlockSpec((D, BF), lambda j: (0, j)),
            pl.BlockSpec((1, BF), lambda j: (0, j)),
        ],
        out_specs=pl.BlockSpec((B, BF), lambda j: (0, j)),
        out_shape=jax.ShapeDtypeStruct((B, F), jnp.float32),
    )(x, W_orig, b_orig.reshape(1, F))

    sk = pl.pallas_call(
        _policy_body,
        grid=(nblk,),
        in_specs=[
            pl.BlockSpec((B, F), lambda j: (0, 0)),
            pl.BlockSpec((F, BF), lambda j: (0, j)),
            pl.BlockSpec((1, BF), lambda j: (0, j)),
            pl.BlockSpec((1, BF), lambda j: (0, j)),
        ],
        out_specs=pl.BlockSpec((B, BF), lambda j: (0, j)),
        out_shape=jax.ShapeDtypeStruct((B, F), jnp.int32),
    )(y, W_policy, b_policy.reshape(1, F), is_sel)

    out = pl.pallas_call(
        functools.partial(_select_blend_body, kth=kth),
        out_shape=jax.ShapeDtypeStruct((B, F), jnp.float32),
    )(sk, y, ground_truth)

    return out


# speedup vs baseline: 2.1107x; 1.0157x over previous
import functools
import math

import jax
import jax.numpy as jnp
from jax import lax
from jax.experimental import pallas as pl
from jax.experimental.pallas import tpu as pltpu
from jax.experimental.pallas import tpu_sc as plsc

_QUANTILE = 0.7


def _build_is_sel(sel_idx, F):
    K = sel_idx.shape[0]
    mesh = plsc.VectorSubcoreMesh(core_axis_name="c", subcore_axis_name="s")

    @functools.partial(
        pl.kernel,
        mesh=mesh,
        compiler_params=pltpu.CompilerParams(needs_layout_passes=False),
        out_type=jax.ShapeDtypeStruct((F,), jnp.int32),
        scratch_types=[
            pltpu.VMEM((K,), jnp.int32),
            pltpu.VMEM((F,), jnp.int32),
        ],
    )
    def sc_scatter(idx_hbm, out_hbm, idx_v, flags_v):
        c = lax.axis_index("c")
        s = lax.axis_index("s")

        @pl.when(jnp.logical_and(c == 0, s == 0))
        def _():
            pltpu.sync_copy(idx_hbm, idx_v)
            zeros = jnp.zeros((16,), jnp.int32)

            def zbody(i, carry):
                flags_v[pl.ds(i * 16, 16)] = zeros
                return carry

            lax.fori_loop(0, F // 16, zbody, 0)
            ones = jnp.ones((16,), jnp.int32)

            def sbody(i, carry):
                iv = idx_v[pl.ds(i * 16, 16)]
                plsc.store_scatter(flags_v, [iv], ones)
                return carry

            lax.fori_loop(0, K // 16, sbody, 0)
            pltpu.sync_copy(flags_v, out_hbm)

    return sc_scatter(sel_idx)


def _fused_body(
    x_ref, wo_ref, wp_ref, bo_ref, bp_ref, sel_ref, gt_ref, o_ref,
    y_s, *, kth, BF, nblk,
):
    j = pl.program_id(0)

    @pl.when(j < nblk)
    def _phase_y():
        col = pl.multiple_of(j * BF, BF)
        y_s[:, pl.ds(col, BF)] = (
            jnp.dot(x_ref[...], wo_ref[...], preferred_element_type=jnp.float32)
            + bo_ref[:, pl.ds(col, BF)]
        )

    @pl.when(jnp.logical_and(j >= nblk, j < 2 * nblk))
    def _phase_z():
        col = pl.multiple_of((j - nblk) * BF, BF)
        z = (
            jnp.dot(y_s[...], wp_ref[...], preferred_element_type=jnp.float32)
            + bp_ref[:, pl.ds(col, BF)]
        )
        bits = lax.bitcast_convert_type(z, jnp.int32)
        skey = jnp.where(bits < 0, bits ^ jnp.int32(0x7FFFFFFF), bits)
        sk_blk = jnp.where(
            sel_ref[:, pl.ds(col, BF)] != 0, skey, jnp.int32(2**31 - 1)
        )
        o_ref[:, pl.ds(col, BF)] = lax.bitcast_convert_type(sk_blk, jnp.float32)

    @pl.when(j == 2 * nblk)
    def _phase_select():
        sk = lax.bitcast_convert_type(o_ref[...], jnp.int32)
        rows = sk.shape[0]
        P0 = jnp.full((rows, 1), jnp.int32(-(2**31)))

        def body(i, P):
            T = P + (jnp.int32(1) << (jnp.int32(31) - i.astype(jnp.int32)))
            cnt = jnp.sum((sk < T).astype(jnp.int32), axis=1, keepdims=True)
            return jnp.where(cnt >= kth, P, T)

        P = lax.fori_loop(0, 32, body, P0)
        o_ref[...] = jnp.where(sk > P, y_s[...], gt_ref[...])


def kernel(x, W_orig, b_orig, W_policy, b_policy, ground_truth, sel_idx):
    B, D = x.shape
    F = W_orig.shape[1]
    K = sel_idx.shape[0]
    kth = int(max(1, min(K, 1 + math.floor(_QUANTILE * (K - 1)))))
    BF = 1024
    nblk = F // BF

    is_sel = _build_is_sel(sel_idx, F).reshape(1, F)

    out = pl.pallas_call(
        functools.partial(_fused_body, kth=kth, BF=BF, nblk=nblk),
        grid=(2 * nblk + 1,),
        in_specs=[
            pl.BlockSpec((B, D), lambda j: (0, 0)),
            pl.BlockSpec(
                (D, BF), lambda j: (0, jnp.minimum(j, nblk - 1))
            ),
            pl.BlockSpec(
                (F, BF),
                lambda j: (0, jnp.clip(j - nblk, 0, nblk - 1)),
            ),
            pl.BlockSpec((1, F), lambda j: (0, 0)),
            pl.BlockSpec((1, F), lambda j: (0, 0)),
            pl.BlockSpec((1, F), lambda j: (0, 0)),
            pl.BlockSpec((B, F), lambda j: (0, 0)),
        ],
        out_specs=pl.BlockSpec((B, F), lambda j: (0, 0)),
        out_shape=jax.ShapeDtypeStruct((B, F), jnp.float32),
        scratch_shapes=[
            pltpu.VMEM((B, F), jnp.float32),
        ],
    )(x, W_orig, W_policy, b_orig.reshape(1, F), b_policy.reshape(1, F),
      is_sel, ground_truth)

    return out


# speedup vs baseline: 2.1465x; 1.0169x over previous
import functools
import math

import jax
import jax.numpy as jnp
from jax import lax
from jax.experimental import pallas as pl
from jax.experimental.pallas import tpu as pltpu
from jax.experimental.pallas import tpu_sc as plsc

_QUANTILE = 0.7


def _build_is_sel(sel_idx, F):
    K = sel_idx.shape[0]
    mesh = plsc.VectorSubcoreMesh(core_axis_name="c", subcore_axis_name="s")

    @functools.partial(
        pl.kernel,
        mesh=mesh,
        compiler_params=pltpu.CompilerParams(needs_layout_passes=False),
        out_type=jax.ShapeDtypeStruct((F,), jnp.int32),
        scratch_types=[
            pltpu.VMEM((K,), jnp.int32),
            pltpu.VMEM((F,), jnp.int32),
        ],
    )
    def sc_scatter(idx_hbm, out_hbm, idx_v, flags_v):
        c = lax.axis_index("c")
        s = lax.axis_index("s")

        @pl.when(jnp.logical_and(c == 0, s == 0))
        def _():
            pltpu.sync_copy(idx_hbm, idx_v)
            zeros = jnp.zeros((16,), jnp.int32)

            def zbody(i, carry):
                flags_v[pl.ds(i * 16, 16)] = zeros
                return carry

            lax.fori_loop(0, F // 16, zbody, 0)
            ones = jnp.ones((16,), jnp.int32)

            def sbody(i, carry):
                iv = idx_v[pl.ds(i * 16, 16)]
                plsc.store_scatter(flags_v, [iv], ones)
                return carry

            lax.fori_loop(0, K // 16, sbody, 0)
            pltpu.sync_copy(flags_v, out_hbm)

    return sc_scatter(sel_idx)


def _fused_body(
    x_ref, wo_ref, wp_ref, bo_ref, bp_ref, sel_ref, gt_ref, o_ref,
    y_s, *, kth, BFO, BFP, nblko, nblkp,
):
    j = pl.program_id(0)

    @pl.when(j < nblko)
    def _phase_y():
        col = pl.multiple_of(j * BFO, BFO)
        y_s[:, pl.ds(col, BFO)] = (
            jnp.dot(x_ref[...], wo_ref[...], preferred_element_type=jnp.float32)
            + bo_ref[:, pl.ds(col, BFO)]
        )

    @pl.when(jnp.logical_and(j >= nblko, j < nblko + nblkp))
    def _phase_z():
        col = pl.multiple_of((j - nblko) * BFP, BFP)
        z = (
            jnp.dot(y_s[...], wp_ref[...], preferred_element_type=jnp.float32)
            + bp_ref[:, pl.ds(col, BFP)]
        )
        bits = lax.bitcast_convert_type(z, jnp.int32)
        skey = jnp.where(bits < 0, bits ^ jnp.int32(0x7FFFFFFF), bits)
        sk_blk = jnp.where(
            sel_ref[:, pl.ds(col, BFP)] != 0, skey, jnp.int32(2**31 - 1)
        )
        o_ref[:, pl.ds(col, BFP)] = lax.bitcast_convert_type(sk_blk, jnp.float32)

    @pl.when(j == nblko + nblkp)
    def _phase_select():
        sk = lax.bitcast_convert_type(o_ref[...], jnp.int32)
        rows = sk.shape[0]
        P0 = jnp.full((rows, 1), jnp.int32(-(2**31)))

        def body(i, P):
            T = P + (jnp.int32(1) << (jnp.int32(31) - i.astype(jnp.int32)))
            cnt = jnp.sum((sk < T).astype(jnp.int32), axis=1, keepdims=True)
            return jnp.where(cnt >= kth, P, T)

        P = lax.fori_loop(0, 32, body, P0)
        o_ref[...] = jnp.where(sk > P, y_s[...], gt_ref[...])


def kernel(x, W_orig, b_orig, W_policy, b_policy, ground_truth, sel_idx):
    B, D = x.shape
    F = W_orig.shape[1]
    K = sel_idx.shape[0]
    kth = int(max(1, min(K, 1 + math.floor(_QUANTILE * (K - 1)))))
    BFO = 2048
    BFP = 512
    nblko = F // BFO
    nblkp = F // BFP

    is_sel = _build_is_sel(sel_idx, F).reshape(1, F)

    out = pl.pallas_call(
        functools.partial(
            _fused_body, kth=kth, BFO=BFO, BFP=BFP, nblko=nblko, nblkp=nblkp
        ),
        grid=(nblko + nblkp + 1,),
        in_specs=[
            pl.BlockSpec((B, D), lambda j: (0, 0)),
            pl.BlockSpec(
                (D, BFO), lambda j: (0, jnp.minimum(j, nblko - 1))
            ),
            pl.BlockSpec(
                (F, BFP),
                lambda j: (0, jnp.clip(j - nblko, 0, nblkp - 1)),
            ),
            pl.BlockSpec((1, F), lambda j: (0, 0)),
            pl.BlockSpec((1, F), lambda j: (0, 0)),
            pl.BlockSpec((1, F), lambda j: (0, 0)),
            pl.BlockSpec((B, F), lambda j: (0, 0)),
        ],
        out_specs=pl.BlockSpec((B, F), lambda j: (0, 0)),
        out_shape=jax.ShapeDtypeStruct((B, F), jnp.float32),
        scratch_shapes=[
            pltpu.VMEM((B, F), jnp.float32),
        ],
    )(x, W_orig, W_policy, b_orig.reshape(1, F), b_policy.reshape(1, F),
      is_sel, ground_truth)

    return out


# speedup vs baseline: 2.2065x; 1.0280x over previous
import functools
import math

import jax
import jax.numpy as jnp
from jax import lax
from jax.experimental import pallas as pl
from jax.experimental.pallas import tpu as pltpu
from jax.experimental.pallas import tpu_sc as plsc

_QUANTILE = 0.7


def _build_is_sel(sel_idx, F):
    K = sel_idx.shape[0]
    mesh = plsc.VectorSubcoreMesh(core_axis_name="c", subcore_axis_name="s")

    @functools.partial(
        pl.kernel,
        mesh=mesh,
        compiler_params=pltpu.CompilerParams(needs_layout_passes=False),
        out_type=jax.ShapeDtypeStruct((F,), jnp.int32),
        scratch_types=[
            pltpu.VMEM((K,), jnp.int32),
            pltpu.VMEM((F,), jnp.int32),
        ],
    )
    def sc_scatter(idx_hbm, out_hbm, idx_v, flags_v):
        c = lax.axis_index("c")
        s = lax.axis_index("s")

        @pl.when(jnp.logical_and(c == 0, s == 0))
        def _():
            pltpu.sync_copy(idx_hbm, idx_v)
            zeros = jnp.zeros((16,), jnp.int32)

            def zbody(i, carry):
                flags_v[pl.ds(i * 16, 16)] = zeros
                return carry

            lax.fori_loop(0, F // 16, zbody, 0)
            ones = jnp.ones((16,), jnp.int32)

            def sbody(i, carry):
                iv = idx_v[pl.ds(i * 16, 16)]
                plsc.store_scatter(flags_v, [iv], ones)
                return carry

            lax.fori_loop(0, K // 16, sbody, 0)
            pltpu.sync_copy(flags_v, out_hbm)

    return sc_scatter(sel_idx)


def _mm_body(
    x_ref, wo_ref, wp_ref, bo_ref, bp_ref, y_ref, sk_ref,
    *, BFO, BFP, nblko, nblkp,
):
    j = pl.program_id(0)

    @pl.when(j < nblko)
    def _phase_y():
        col = pl.multiple_of(j * BFO, BFO)
        y_ref[:, pl.ds(col, BFO)] = (
            jnp.dot(x_ref[...], wo_ref[...], preferred_element_type=jnp.float32)
            + bo_ref[:, pl.ds(col, BFO)]
        )

    @pl.when(j >= nblko)
    def _phase_z():
        col = pl.multiple_of((j - nblko) * BFP, BFP)
        z = (
            jnp.dot(y_ref[...], wp_ref[...], preferred_element_type=jnp.float32)
            + bp_ref[:, pl.ds(col, BFP)]
        )
        bits = lax.bitcast_convert_type(z, jnp.int32)
        sk_ref[:, pl.ds(col, BFP)] = jnp.where(
            bits < 0, bits ^ jnp.int32(0x7FFFFFFF), bits
        )


def _select_body(sk_ref, sel_ref, y_ref, gt_ref, o_ref, *, kth):
    sk = jnp.where(sel_ref[...] != 0, sk_ref[...], jnp.int32(2**31 - 1))
    rows = sk.shape[0]
    P0 = jnp.full((rows, 1), jnp.int32(-(2**31)))

    def body(i, P):
        T = P + (jnp.int32(1) << (jnp.int32(31) - i.astype(jnp.int32)))
        cnt = jnp.sum((sk < T).astype(jnp.int32), axis=1, keepdims=True)
        return jnp.where(cnt >= kth, P, T)

    P = lax.fori_loop(0, 32, body, P0)
    o_ref[...] = jnp.where(sk > P, y_ref[...], gt_ref[...])


def kernel(x, W_orig, b_orig, W_policy, b_policy, ground_truth, sel_idx):
    B, D = x.shape
    F = W_orig.shape[1]
    K = sel_idx.shape[0]
    kth = int(max(1, min(K, 1 + math.floor(_QUANTILE * (K - 1)))))
    BFO = 2048
    BFP = 512
    nblko = F // BFO
    nblkp = F // BFP

    is_sel = _build_is_sel(sel_idx, F).reshape(1, F)

    y, sk = pl.pallas_call(
        functools.partial(
            _mm_body, BFO=BFO, BFP=BFP, nblko=nblko, nblkp=nblkp
        ),
        grid=(nblko + nblkp,),
        in_specs=[
            pl.BlockSpec((B, D), lambda j: (0, 0)),
            pl.BlockSpec(
                (D, BFO), lambda j: (0, jnp.minimum(j, nblko - 1))
            ),
            pl.BlockSpec(
                (F, BFP),
                lambda j: (0, jnp.clip(j - nblko, 0, nblkp - 1)),
            ),
            pl.BlockSpec((1, F), lambda j: (0, 0)),
            pl.BlockSpec((1, F), lambda j: (0, 0)),
        ],
        out_specs=[
            pl.BlockSpec((B, F), lambda j: (0, 0)),
            pl.BlockSpec((B, F), lambda j: (0, 0)),
        ],
        out_shape=[
            jax.ShapeDtypeStruct((B, F), jnp.float32),
            jax.ShapeDtypeStruct((B, F), jnp.int32),
        ],
    )(x, W_orig, W_policy, b_orig.reshape(1, F), b_policy.reshape(1, F))

    out = pl.pallas_call(
        functools.partial(_select_body, kth=kth),
        out_shape=jax.ShapeDtypeStruct((B, F), jnp.float32),
    )(sk, is_sel, y, ground_truth)

    return out


# speedup vs baseline: 2.5614x; 1.1608x over previous
import functools
import math

import jax
import jax.numpy as jnp
from jax import lax
from jax.experimental import pallas as pl
from jax.experimental.pallas import tpu as pltpu
from jax.experimental.pallas import tpu_sc as plsc

_QUANTILE = 0.7


def _mm_body(
    x_ref, wo_ref, wp_ref, bo_ref, bp_ref, selidx_ref, y_ref, sk_ref,
    issel_ref, *, BFO, BFP, nblko, nblkp, CH,
):
    j = pl.program_id(0)
    F = y_ref.shape[1]

    @pl.when(j == 0)
    def _build_indicator():
        selc = selidx_ref[...]

        def chunk(ci, carry):
            base = ci * CH
            cols = base + lax.broadcasted_iota(jnp.int32, (1, CH), 1)
            hit = jnp.any(selc == cols, axis=0, keepdims=True)
            issel_ref[:, pl.ds(base, CH)] = hit.astype(jnp.int32)
            return carry

        lax.fori_loop(0, F // CH, chunk, 0)

    @pl.when(j < nblko)
    def _phase_y():
        col = pl.multiple_of(j * BFO, BFO)
        y_ref[:, pl.ds(col, BFO)] = (
            jnp.dot(x_ref[...], wo_ref[...], preferred_element_type=jnp.float32)
            + bo_ref[:, pl.ds(col, BFO)]
        )

    @pl.when(j >= nblko)
    def _phase_z():
        col = pl.multiple_of((j - nblko) * BFP, BFP)
        z = (
            jnp.dot(y_ref[...], wp_ref[...], preferred_element_type=jnp.float32)
            + bp_ref[:, pl.ds(col, BFP)]
        )
        bits = lax.bitcast_convert_type(z, jnp.int32)
        skey = jnp.where(bits < 0, bits ^ jnp.int32(0x7FFFFFFF), bits)
        sk_ref[:, pl.ds(col, BFP)] = jnp.where(
            issel_ref[:, pl.ds(col, BFP)] != 0, skey, jnp.int32(2**31 - 1)
        )


def _select_body(sk_ref, y_ref, gt_ref, o_ref, *, kth):
    sk = sk_ref[...]
    rows = sk.shape[0]
    P0 = jnp.full((rows, 1), jnp.int32(-(2**31)))

    def body(i, P):
        T = P + (jnp.int32(1) << (jnp.int32(31) - i.astype(jnp.int32)))
        cnt = jnp.sum((sk < T).astype(jnp.int32), axis=1, keepdims=True)
        return jnp.where(cnt >= kth, P, T)

    P = lax.fori_loop(0, 32, body, P0)
    o_ref[...] = jnp.where(sk > P, y_ref[...], gt_ref[...])


def kernel(x, W_orig, b_orig, W_policy, b_policy, ground_truth, sel_idx):
    B, D = x.shape
    F = W_orig.shape[1]
    K = sel_idx.shape[0]
    kth = int(max(1, min(K, 1 + math.floor(_QUANTILE * (K - 1)))))
    BFO = 2048
    BFP = 512
    nblko = F // BFO
    nblkp = F // BFP
    CH = 512

    y, sk = pl.pallas_call(
        functools.partial(
            _mm_body, BFO=BFO, BFP=BFP, nblko=nblko, nblkp=nblkp, CH=CH
        ),
        grid=(nblko + nblkp,),
        in_specs=[
            pl.BlockSpec((B, D), lambda j: (0, 0)),
            pl.BlockSpec(
                (D, BFO), lambda j: (0, jnp.minimum(j, nblko - 1))
            ),
            pl.BlockSpec(
                (F, BFP),
                lambda j: (0, jnp.clip(j - nblko, 0, nblkp - 1)),
            ),
            pl.BlockSpec((1, F), lambda j: (0, 0)),
            pl.BlockSpec((1, F), lambda j: (0, 0)),
            pl.BlockSpec((K, 1), lambda j: (0, 0)),
        ],
        out_specs=[
            pl.BlockSpec((B, F), lambda j: (0, 0)),
            pl.BlockSpec((B, F), lambda j: (0, 0)),
        ],
        out_shape=[
            jax.ShapeDtypeStruct((B, F), jnp.float32),
            jax.ShapeDtypeStruct((B, F), jnp.int32),
        ],
        scratch_shapes=[
            pltpu.VMEM((1, F), jnp.int32),
        ],
    )(x, W_orig, W_policy, b_orig.reshape(1, F), b_policy.reshape(1, F),
      sel_idx.reshape(K, 1))

    out = pl.pallas_call(
        functools.partial(_select_body, kth=kth),
        out_shape=jax.ShapeDtypeStruct((B, F), jnp.float32),
    )(sk, y, ground_truth)

    return out


# speedup vs baseline: 2.7028x; 1.0552x over previous
import functools
import math

import jax
import jax.numpy as jnp
from jax import lax
from jax.experimental import pallas as pl
from jax.experimental.pallas import tpu as pltpu
from jax.experimental.pallas import tpu_sc as plsc

_QUANTILE = 0.7


def _mm_body(
    x_ref, wo_ref, wp_ref, bo_ref, bp_ref, selidx_ref, y_ref, sk_ref,
    issel_ref, *, BFO, BFP, nblko, nblkp, CH,
):
    j = pl.program_id(0)
    F = y_ref.shape[1]

    @pl.when(j < F // CH)
    def _build_indicator():
        selc = selidx_ref[...]
        base = pl.multiple_of(j * CH, CH)
        cols = base + lax.broadcasted_iota(jnp.int32, (1, CH), 1)
        hit = jnp.any(selc == cols, axis=0, keepdims=True)
        issel_ref[:, pl.ds(base, CH)] = hit.astype(jnp.int32)

    @pl.when(j < nblko)
    def _phase_y():
        col = pl.multiple_of(j * BFO, BFO)
        y_ref[:, pl.ds(col, BFO)] = (
            jnp.dot(x_ref[...], wo_ref[...], preferred_element_type=jnp.float32)
            + bo_ref[:, pl.ds(col, BFO)]
        )

    @pl.when(j >= nblko)
    def _phase_z():
        col = pl.multiple_of((j - nblko) * BFP, BFP)
        z = (
            jnp.dot(y_ref[...], wp_ref[...], preferred_element_type=jnp.float32)
            + bp_ref[:, pl.ds(col, BFP)]
        )
        bits = lax.bitcast_convert_type(z, jnp.int32)
        skey = jnp.where(bits < 0, bits ^ jnp.int32(0x7FFFFFFF), bits)
        sk_ref[:, pl.ds(col, BFP)] = jnp.where(
            issel_ref[:, pl.ds(col, BFP)] != 0, skey, jnp.int32(2**31 - 1)
        )


def _select_body(sk_ref, y_ref, gt_ref, o_ref, *, kth):
    sk = sk_ref[...]
    rows = sk.shape[0]
    P0 = jnp.full((rows, 1), jnp.int32(-(2**31)))

    def body(i, P):
        T = P + (jnp.int32(1) << (jnp.int32(31) - i.astype(jnp.int32)))
        cnt = jnp.sum((sk < T).astype(jnp.int32), axis=1, keepdims=True)
        return jnp.where(cnt >= kth, P, T)

    P = lax.fori_loop(0, 32, body, P0)
    o_ref[...] = jnp.where(sk > P, y_ref[...], gt_ref[...])


def kernel(x, W_orig, b_orig, W_policy, b_policy, ground_truth, sel_idx):
    B, D = x.shape
    F = W_orig.shape[1]
    K = sel_idx.shape[0]
    kth = int(max(1, min(K, 1 + math.floor(_QUANTILE * (K - 1)))))
    BFO = 2048
    BFP = 512
    nblko = F // BFO
    nblkp = F // BFP
    CH = 512

    y, sk = pl.pallas_call(
        functools.partial(
            _mm_body, BFO=BFO, BFP=BFP, nblko=nblko, nblkp=nblkp, CH=CH
        ),
        grid=(nblko + nblkp,),
        in_specs=[
            pl.BlockSpec((B, D), lambda j: (0, 0)),
            pl.BlockSpec(
                (D, BFO), lambda j: (0, jnp.minimum(j, nblko - 1))
            ),
            pl.BlockSpec(
                (F, BFP),
                lambda j: (0, jnp.clip(j - nblko, 0, nblkp - 1)),
            ),
            pl.BlockSpec((1, F), lambda j: (0, 0)),
            pl.BlockSpec((1, F), lambda j: (0, 0)),
            pl.BlockSpec((K, 1), lambda j: (0, 0)),
        ],
        out_specs=[
            pl.BlockSpec((B, F), lambda j: (0, 0)),
            pl.BlockSpec((B, F), lambda j: (0, 0)),
        ],
        out_shape=[
            jax.ShapeDtypeStruct((B, F), jnp.float32),
            jax.ShapeDtypeStruct((B, F), jnp.int32),
        ],
        scratch_shapes=[
            pltpu.VMEM((1, F), jnp.int32),
        ],
    )(x, W_orig, W_policy, b_orig.reshape(1, F), b_policy.reshape(1, F),
      sel_idx.reshape(K, 1))

    out = pl.pallas_call(
        functools.partial(_select_body, kth=kth),
        out_shape=jax.ShapeDtypeStruct((B, F), jnp.float32),
    )(sk, y, ground_truth)

    return out


# speedup vs baseline: 2.9154x; 1.0786x over previous
import functools
import math

import jax
import jax.numpy as jnp
from jax import lax
from jax.experimental import pallas as pl
from jax.experimental.pallas import tpu as pltpu
from jax.experimental.pallas import tpu_sc as plsc

_QUANTILE = 0.7


def _fused_body(
    x_ref, wo_ref, wp_ref, bo_ref, bp_ref, selidx_ref, gt_ref, o_ref,
    y_s, issel_s, *, kth, BFO, BFP, nblko, nblkp, CH,
):
    j = pl.program_id(0)
    F = y_s.shape[1]

    @pl.when(j < F // CH)
    def _build_indicator():
        selc = selidx_ref[...]
        base = pl.multiple_of(j * CH, CH)
        cols = base + lax.broadcasted_iota(jnp.int32, (1, CH), 1)
        hit = jnp.any(selc == cols, axis=0, keepdims=True)
        issel_s[:, pl.ds(base, CH)] = hit.astype(jnp.int32)

    @pl.when(j < nblko)
    def _phase_y():
        col = pl.multiple_of(j * BFO, BFO)
        y_s[:, pl.ds(col, BFO)] = (
            jnp.dot(x_ref[...], wo_ref[...], preferred_element_type=jnp.float32)
            + bo_ref[:, pl.ds(col, BFO)]
        )

    @pl.when(jnp.logical_and(j >= nblko, j < nblko + nblkp))
    def _phase_z():
        col = pl.multiple_of((j - nblko) * BFP, BFP)
        z = (
            jnp.dot(y_s[...], wp_ref[...], preferred_element_type=jnp.float32)
            + bp_ref[:, pl.ds(col, BFP)]
        )
        bits = lax.bitcast_convert_type(z, jnp.int32)
        skey = jnp.where(bits < 0, bits ^ jnp.int32(0x7FFFFFFF), bits)
        sk_blk = jnp.where(
            issel_s[:, pl.ds(col, BFP)] != 0, skey, jnp.int32(2**31 - 1)
        )
        o_ref[:, pl.ds(col, BFP)] = lax.bitcast_convert_type(sk_blk, jnp.float32)

    @pl.when(j == nblko + nblkp)
    def _phase_select():
        sk = lax.bitcast_convert_type(o_ref[...], jnp.int32)
        rows = sk.shape[0]
        P0 = jnp.full((rows, 1), jnp.int32(-(2**31)))

        def body(i, P):
            T = P + (jnp.int32(1) << (jnp.int32(31) - i.astype(jnp.int32)))
            cnt = jnp.sum((sk < T).astype(jnp.int32), axis=1, keepdims=True)
            return jnp.where(cnt >= kth, P, T)

        P = lax.fori_loop(0, 32, body, P0)
        o_ref[...] = jnp.where(sk > P, y_s[...], gt_ref[...])


def kernel(x, W_orig, b_orig, W_policy, b_policy, ground_truth, sel_idx):
    B, D = x.shape
    F = W_orig.shape[1]
    K = sel_idx.shape[0]
    kth = int(max(1, min(K, 1 + math.floor(_QUANTILE * (K - 1)))))
    BFO = 2048
    BFP = 512
    nblko = F // BFO
    nblkp = F // BFP
    CH = 512

    out = pl.pallas_call(
        functools.partial(
            _fused_body, kth=kth, BFO=BFO, BFP=BFP,
            nblko=nblko, nblkp=nblkp, CH=CH,
        ),
        grid=(nblko + nblkp + 1,),
        in_specs=[
            pl.BlockSpec((B, D), lambda j: (0, 0)),
            pl.BlockSpec(
                (D, BFO), lambda j: (0, jnp.minimum(j, nblko - 1))
            ),
            pl.BlockSpec(
                (F, BFP),
                lambda j: (0, jnp.clip(j - nblko, 0, nblkp - 1)),
            ),
            pl.BlockSpec((1, F), lambda j: (0, 0)),
            pl.BlockSpec((1, F), lambda j: (0, 0)),
            pl.BlockSpec((K, 1), lambda j: (0, 0)),
            pl.BlockSpec((B, F), lambda j: (0, 0)),
        ],
        out_specs=pl.BlockSpec((B, F), lambda j: (0, 0)),
        out_shape=jax.ShapeDtypeStruct((B, F), jnp.float32),
        scratch_shapes=[
            pltpu.VMEM((B, F), jnp.float32),
            pltpu.VMEM((1, F), jnp.int32),
        ],
    )(x, W_orig, W_policy, b_orig.reshape(1, F), b_policy.reshape(1, F),
      sel_idx.reshape(K, 1), ground_truth)

    return out
